# baseline (speedup 1.0000x reference)
import functools

import jax
import jax.numpy as jnp
from jax import lax
from jax.experimental import pallas as pl
from jax.experimental.pallas import tpu as pltpu
from jax.experimental.pallas import tpu_sc as plsc

N = 10000
E = 320000
D = 128
R = 8
EPS = 1e-5

DP = 144
NC = 2
NS = 16
EPT = E // NS
MC = 4000
NMC = EPT // MC
RPC = R // NC
G = 32
KF = 4
ROWS_PT = 641
NPAD = NS * ROWS_PT
TRASH = N
TP = MC + 2 * KF * G
CAP = TP + 16


def _sc_agg_body(hpad, packed, zeros, out, acc, metam,
                 gidx, sidx, rows0, rows1, gsem0, gsem1, ssem0, ssem1):
    c = lax.axis_index("c")
    s = lax.axis_index("s")

    pltpu.sync_copy(zeros, acc.at[pl.ds(s * ROWS_PT, ROWS_PT)])
    plsc.subcore_barrier()

    @pl.loop(0, RPC)
    def _per_relation(rl):
        r = c * RPC + rl

        def per_chunk(m, _):
            with jax.named_scope("meta_dma"):
                base = s * EPT + m * MC
                pltpu.sync_copy(packed.at[pl.ds(base, MC)], metam)

            def scan_group(i, off):
                p = metam[pl.ds(i * 16, 16)]
                t = p & 7
                dv = (p >> 3) & 16383
                sv = p >> 17
                d = t ^ r
                mi = 1 - ((d + 7) >> 3)
                cum = plsc.cumsum(mi)
                pos = (off + cum - 1 - TP) * mi + TP
                plsc.store_scatter(gidx, [pos], sv)
                plsc.store_scatter(sidx, [pos], dv)
                return off + cum[15]

            with jax.named_scope("scan"):
                n_c = lax.fori_loop(0, MC // 16, scan_group, jnp.int32(0))

            pad_end = ((n_c + KF * G - 1) // (KF * G)) * (KF * G)

            def pad_group(k, _):
                j = n_c + k * 16
                gidx[pl.ds(j, 16)] = jnp.zeros((16,), jnp.int32)
                sidx[pl.ds(j, 16)] = TRASH + s * 16 + lax.iota(jnp.int32, 16)
                return 0

            lax.fori_loop(0, (pad_end - n_c + 15) // 16, pad_group, 0)

            def move_chunk(q, _):
                base = q * KF * G
                for b in range(KF):
                    pltpu.async_copy(
                        hpad.at[gidx.at[pl.ds(base + b * G, G)]],
                        rows0.at[pl.ds(b * G, G)], gsem0)
                for b in range(KF):
                    pltpu.make_async_copy(
                        hpad.at[gidx.at[pl.ds(base + b * G, G)]],
                        rows0.at[pl.ds(b * G, G)], gsem0).wait()
                for b in range(KF):
                    pltpu.async_copy(
                        rows0.at[pl.ds(b * G, G)],
                        acc.at[sidx.at[pl.ds(base + b * G, G)]], ssem0,
                        add=True)
                for b in range(KF):
                    pltpu.make_async_copy(
                        rows0.at[pl.ds(b * G, G)],
                        acc.at[sidx.at[pl.ds(base + b * G, G)]], ssem0).wait()
                return 0

            with jax.named_scope("move"):
                lax.fori_loop(0, pad_end // (KF * G), move_chunk, 0)
            return 0

        lax.fori_loop(0, NMC, per_chunk, 0)
        plsc.subcore_barrier()

        with jax.named_scope("flush"):
            rowbase = s * ROWS_PT
            pltpu.sync_copy(acc.at[pl.ds(rowbase, ROWS_PT)],
                            out.at[r, pl.ds(rowbase, ROWS_PT)])
            pltpu.sync_copy(zeros, acc.at[pl.ds(rowbase, ROWS_PT)])
            plsc.subcore_barrier()


@functools.cache
def _make_sc_agg():
    return pl.kernel(
        _sc_agg_body,
        out_type=jax.ShapeDtypeStruct((R, NPAD, DP), jnp.float32),
        mesh=plsc.VectorSubcoreMesh(core_axis_name="c", subcore_axis_name="s",
                                    num_cores=NC, num_subcores=NS),
        scratch_types=[
            pltpu.MemorySpace.VMEM_SHARED((NPAD, DP), jnp.float32),
            pltpu.VMEM((MC,), jnp.int32),
            pltpu.VMEM((CAP,), jnp.int32),
            pltpu.VMEM((CAP,), jnp.int32),
            pltpu.VMEM((KF * G, DP), jnp.float32),
            pltpu.VMEM((G, DP), jnp.float32),
            pltpu.SemaphoreType.DMA,
            pltpu.SemaphoreType.DMA,
            pltpu.SemaphoreType.DMA,
            pltpu.SemaphoreType.DMA,
        ],
        compiler_params=pltpu.CompilerParams(needs_layout_passes=False,
                                             use_tc_tiling_on_sc=False),
    )


BN = 2000


def _tc_layer_body(first, agg_ref, hin_ref, root_ref, w_ref, b_ref, g_ref,
                   beta_ref, out_ref):
    hin = hin_ref[:, :D]
    acc = jnp.dot(hin, root_ref[...], preferred_element_type=jnp.float32)
    acc = acc + b_ref[...]
    a = agg_ref[...]
    cnt = a[:, :, D:D + 1]
    inv = 1.0 / jnp.maximum(cnt, 1.0)
    for r in range(R):
        mean = a[r, :, :D] * inv[r]
        acc = acc + jnp.dot(mean, w_ref[r], preferred_element_type=jnp.float32)
    if first:
        acc = jnp.where(acc > 0, acc, 0.2 * acc)
    mu = jnp.mean(acc, axis=1, keepdims=True)
    var = jnp.mean((acc - mu) ** 2, axis=1, keepdims=True)
    h = (acc - mu) * lax.rsqrt(var + EPS) * g_ref[...] + beta_ref[...]
    if first:
        hp = jnp.pad(h, ((0, 0), (0, DP - D)))
        lane = lax.broadcasted_iota(jnp.int32, (BN, DP), 1)
        out_ref[...] = hp + jnp.where(lane == D, 1.0, 0.0)
    else:
        out_ref[...] = h


def _tc_layer(first, agg, hin, root, w, b, g, beta):
    out_w = DP if first else D
    return pl.pallas_call(
        functools.partial(_tc_layer_body, first),
        grid=(N // BN,),
        in_specs=[
            pl.BlockSpec((R, BN, DP), lambda i: (0, i, 0)),
            pl.BlockSpec((BN, DP), lambda i: (i, 0)),
            pl.BlockSpec((D, D), lambda i: (0, 0)),
            pl.BlockSpec((R, D, D), lambda i: (0, 0, 0)),
            pl.BlockSpec((1, D), lambda i: (0, 0)),
            pl.BlockSpec((1, D), lambda i: (0, 0)),
            pl.BlockSpec((1, D), lambda i: (0, 0)),
        ],
        out_specs=pl.BlockSpec((BN, out_w), lambda i: (i, 0)),
        out_shape=jax.ShapeDtypeStruct((N, out_w), jnp.float32),
    )(agg, hin, root, w, b, g, beta)


def kernel(x, edge_index, edge_type, W1, root1, b1, g1, beta1,
           W2, root2, b2, g2, beta2):
    src = edge_index[0]
    dst = edge_index[1]
    packed = (src << 17) | (dst << 3) | edge_type
    ones_col = jnp.concatenate(
        [jnp.zeros((N, D), jnp.float32),
         jnp.ones((N, 1), jnp.float32),
         jnp.zeros((N, DP - D - 1), jnp.float32)], axis=1)
    xpad = jnp.pad(x, ((0, 0), (0, DP - D))) + ones_col
    zeros = jnp.zeros((ROWS_PT, DP), jnp.float32)

    _sc_agg = _make_sc_agg()
    agg1 = _sc_agg(xpad, packed, zeros)
    h1pad = _tc_layer(True, agg1, xpad, root1, W1, b1.reshape(1, D),
                      g1.reshape(1, D), beta1.reshape(1, D))
    agg2 = _sc_agg(h1pad, packed, zeros)
    h2 = _tc_layer(False, agg2, h1pad, root2, W2, b2.reshape(1, D),
                   g2.reshape(1, D), beta2.reshape(1, D))
    return h2


# speedup vs baseline: 1.8679x; 1.8679x over previous
import functools

import jax
import jax.numpy as jnp
from jax import lax
from jax.experimental import pallas as pl
from jax.experimental.pallas import tpu as pltpu
from jax.experimental.pallas import tpu_sc as plsc

N = 10000
E = 320000
D = 128
R = 8
EPS = 1e-5

DP = 144
NC = 2
NS = 16
EPT = E // NS
MC = 4000
NMC = EPT // MC
RPC = R // NC
G = 32
KF = 1
ROWS_PT = 641
NPAD = NS * ROWS_PT
TRASH = N
TP = MC + 2 * KF * G
CAP = TP + 16


def _sc_agg_body(hpad, packed, zeros, out, acc, metam,
                 gidx, sidx, rows0, rows1, gsem0, gsem1, ssem0, ssem1):
    c = lax.axis_index("c")
    s = lax.axis_index("s")

    pltpu.sync_copy(zeros, acc.at[pl.ds(s * ROWS_PT, ROWS_PT)])
    plsc.subcore_barrier()

    @pl.loop(0, RPC)
    def _per_relation(rl):
        r = c * RPC + rl

        def per_chunk(m, _):
            with jax.named_scope("meta_dma"):
                base = s * EPT + m * MC
                pltpu.sync_copy(packed.at[pl.ds(base, MC)], metam)

            def scan_group(i, off):
                p = metam[pl.ds(i * 16, 16)]
                t = p & 7
                dv = (p >> 3) & 16383
                sv = p >> 17
                d = t ^ r
                mi = 1 - ((d + 7) >> 3)
                cum = plsc.cumsum(mi)
                pos = (off + cum - 1 - TP) * mi + TP
                plsc.store_scatter(gidx, [pos], sv)
                plsc.store_scatter(sidx, [pos], dv)
                return off + cum[15]

            with jax.named_scope("scan"):
                n_c = lax.fori_loop(0, MC // 16, scan_group, jnp.int32(0))

            pad_end = ((n_c + KF * G - 1) // (KF * G)) * (KF * G)

            def pad_group(k, _):
                j = n_c + k * 16
                gidx[pl.ds(j, 16)] = jnp.zeros((16,), jnp.int32)
                sidx[pl.ds(j, 16)] = TRASH + s * 16 + lax.iota(jnp.int32, 16)
                return 0

            lax.fori_loop(0, (pad_end - n_c + 15) // 16, pad_group, 0)

            def move_chunk(q, _):
                base = q * KF * G
                for b in range(KF):
                    pltpu.async_copy(
                        hpad.at[gidx.at[pl.ds(base + b * G, G)]],
                        rows0.at[pl.ds(b * G, G)], gsem0)
                for b in range(KF):
                    pltpu.make_async_copy(
                        hpad.at[gidx.at[pl.ds(base + b * G, G)]],
                        rows0.at[pl.ds(b * G, G)], gsem0).wait()
                for b in range(KF):
                    pltpu.async_copy(
                        rows0.at[pl.ds(b * G, G)],
                        acc.at[sidx.at[pl.ds(base + b * G, G)]], ssem0,
                        add=True)
                for b in range(KF):
                    pltpu.make_async_copy(
                        rows0.at[pl.ds(b * G, G)],
                        acc.at[sidx.at[pl.ds(base + b * G, G)]], ssem0).wait()
                return 0

            with jax.named_scope("move"):
                lax.fori_loop(0, pad_end // (KF * G), move_chunk, 0)
            return 0

        lax.fori_loop(0, NMC, per_chunk, 0)
        plsc.subcore_barrier()

        with jax.named_scope("flush"):
            rowbase = s * ROWS_PT
            pltpu.sync_copy(acc.at[pl.ds(rowbase, ROWS_PT)],
                            out.at[r, pl.ds(rowbase, ROWS_PT)])
            pltpu.sync_copy(zeros, acc.at[pl.ds(rowbase, ROWS_PT)])
            plsc.subcore_barrier()


@functools.cache
def _make_sc_agg():
    return pl.kernel(
        _sc_agg_body,
        out_type=jax.ShapeDtypeStruct((R, NPAD, DP), jnp.float32),
        mesh=plsc.VectorSubcoreMesh(core_axis_name="c", subcore_axis_name="s",
                                    num_cores=NC, num_subcores=NS),
        scratch_types=[
            pltpu.MemorySpace.VMEM_SHARED((NPAD, DP), jnp.float32),
            pltpu.VMEM((MC,), jnp.int32),
            pltpu.VMEM((CAP,), jnp.int32),
            pltpu.VMEM((CAP,), jnp.int32),
            pltpu.VMEM((KF * G, DP), jnp.float32),
            pltpu.VMEM((G, DP), jnp.float32),
            pltpu.SemaphoreType.DMA,
            pltpu.SemaphoreType.DMA,
            pltpu.SemaphoreType.DMA,
            pltpu.SemaphoreType.DMA,
        ],
        compiler_params=pltpu.CompilerParams(needs_layout_passes=False,
                                             use_tc_tiling_on_sc=False),
    )


BN = 2000


def _tc_layer_body(first, agg_ref, hin_ref, root_ref, w_ref, b_ref, g_ref,
                   beta_ref, out_ref):
    hin = hin_ref[:, :D]
    acc = jnp.dot(hin, root_ref[...], preferred_element_type=jnp.float32)
    acc = acc + b_ref[...]
    a = agg_ref[...]
    cnt = a[:, :, D:D + 1]
    inv = 1.0 / jnp.maximum(cnt, 1.0)
    for r in range(R):
        mean = a[r, :, :D] * inv[r]
        acc = acc + jnp.dot(mean, w_ref[r], preferred_element_type=jnp.float32)
    if first:
        acc = jnp.where(acc > 0, acc, 0.2 * acc)
    mu = jnp.mean(acc, axis=1, keepdims=True)
    var = jnp.mean((acc - mu) ** 2, axis=1, keepdims=True)
    h = (acc - mu) * lax.rsqrt(var + EPS) * g_ref[...] + beta_ref[...]
    if first:
        hp = jnp.pad(h, ((0, 0), (0, DP - D)))
        lane = lax.broadcasted_iota(jnp.int32, (BN, DP), 1)
        out_ref[...] = hp + jnp.where(lane == D, 1.0, 0.0)
    else:
        out_ref[...] = h


def _tc_layer(first, agg, hin, root, w, b, g, beta):
    out_w = DP if first else D
    return pl.pallas_call(
        functools.partial(_tc_layer_body, first),
        grid=(N // BN,),
        in_specs=[
            pl.BlockSpec((R, BN, DP), lambda i: (0, i, 0)),
            pl.BlockSpec((BN, DP), lambda i: (i, 0)),
            pl.BlockSpec((D, D), lambda i: (0, 0)),
            pl.BlockSpec((R, D, D), lambda i: (0, 0, 0)),
            pl.BlockSpec((1, D), lambda i: (0, 0)),
            pl.BlockSpec((1, D), lambda i: (0, 0)),
            pl.BlockSpec((1, D), lambda i: (0, 0)),
        ],
        out_specs=pl.BlockSpec((BN, out_w), lambda i: (i, 0)),
        out_shape=jax.ShapeDtypeStruct((N, out_w), jnp.float32),
    )(agg, hin, root, w, b, g, beta)


def kernel(x, edge_index, edge_type, W1, root1, b1, g1, beta1,
           W2, root2, b2, g2, beta2):
    src = edge_index[0]
    dst = edge_index[1]
    packed = (src << 17) | (dst << 3) | edge_type
    ones_col = jnp.concatenate(
        [jnp.zeros((N, D), jnp.float32),
         jnp.ones((N, 1), jnp.float32),
         jnp.zeros((N, DP - D - 1), jnp.float32)], axis=1)
    xpad = jnp.pad(x, ((0, 0), (0, DP - D))) + ones_col
    zeros = jnp.zeros((ROWS_PT, DP), jnp.float32)

    _sc_agg = _make_sc_agg()
    agg1 = _sc_agg(xpad, packed, zeros)
    h1pad = _tc_layer(True, agg1, xpad, root1, W1, b1.reshape(1, D),
                      g1.reshape(1, D), beta1.reshape(1, D))
    agg2 = _sc_agg(h1pad, packed, zeros)
    h2 = _tc_layer(False, agg2, h1pad, root2, W2, b2.reshape(1, D),
                   g2.reshape(1, D), beta2.reshape(1, D))
    return h2


# speedup vs baseline: 1.9339x; 1.0354x over previous
import functools

import jax
import jax.numpy as jnp
from jax import lax
from jax.experimental import pallas as pl
from jax.experimental.pallas import tpu as pltpu
from jax.experimental.pallas import tpu_sc as plsc

N = 10000
E = 320000
D = 128
R = 8
EPS = 1e-5

DP = 144
NC = 2
NS = 16
EPT = E // NS
MC = 4000
NMC = EPT // MC
RPC = R // NC
G = 32
KF = 1
ROWS_PT = 641
NPAD = NS * ROWS_PT
TRASH = N
TP = MC + 2 * KF * G
CAP = TP + 16


def _sc_agg_body(hpad, packed, zeros, outs, outc, acc, metam,
                 gidx, sidx, rows0, rows1, gsem0, gsem1, ssem0, ssem1):
    c = lax.axis_index("c")
    s = lax.axis_index("s")

    pltpu.sync_copy(zeros, acc.at[pl.ds(s * ROWS_PT, ROWS_PT)])
    plsc.subcore_barrier()

    @pl.loop(0, RPC)
    def _per_relation(rl):
        r = c * RPC + rl

        def per_chunk(m, _):
            with jax.named_scope("meta_dma"):
                base = s * EPT + m * MC
                pltpu.sync_copy(packed.at[pl.ds(base, MC)], metam)

            def scan_group(i, off):
                p = metam[pl.ds(i * 16, 16)]
                t = p & 7
                dv = (p >> 3) & 16383
                sv = p >> 17
                d = t ^ r
                mi = 1 - ((d + 7) >> 3)
                cum = plsc.cumsum(mi)
                pos = (off + cum - 1 - TP) * mi + TP
                plsc.store_scatter(gidx, [pos], sv)
                plsc.store_scatter(sidx, [pos], dv)
                return off + cum[15]

            with jax.named_scope("scan"):
                n_c = lax.fori_loop(0, MC // 16, scan_group, jnp.int32(0))

            pad_end = ((n_c + KF * G - 1) // (KF * G)) * (KF * G)

            def pad_group(k, _):
                j = n_c + k * 16
                gidx[pl.ds(j, 16)] = jnp.zeros((16,), jnp.int32)
                sidx[pl.ds(j, 16)] = TRASH + s * 16 + lax.iota(jnp.int32, 16)
                return 0

            lax.fori_loop(0, (pad_end - n_c + 15) // 16, pad_group, 0)

            def move_chunk(q, _):
                base = q * KF * G
                for b in range(KF):
                    pltpu.async_copy(
                        hpad.at[gidx.at[pl.ds(base + b * G, G)]],
                        rows0.at[pl.ds(b * G, G)], gsem0)
                for b in range(KF):
                    pltpu.make_async_copy(
                        hpad.at[gidx.at[pl.ds(base + b * G, G)]],
                        rows0.at[pl.ds(b * G, G)], gsem0).wait()
                for b in range(KF):
                    pltpu.async_copy(
                        rows0.at[pl.ds(b * G, G)],
                        acc.at[sidx.at[pl.ds(base + b * G, G)]], ssem0,
                        add=True)
                for b in range(KF):
                    pltpu.make_async_copy(
                        rows0.at[pl.ds(b * G, G)],
                        acc.at[sidx.at[pl.ds(base + b * G, G)]], ssem0).wait()
                return 0

            with jax.named_scope("move"):
                lax.fori_loop(0, pad_end // (KF * G), move_chunk, 0)
            return 0

        lax.fori_loop(0, NMC, per_chunk, 0)
        plsc.subcore_barrier()

        with jax.named_scope("flush"):
            rowbase = s * ROWS_PT
            pltpu.sync_copy(acc.at[pl.ds(rowbase, ROWS_PT), pl.ds(0, D)],
                            outs.at[r, pl.ds(rowbase, ROWS_PT)])
            pltpu.sync_copy(acc.at[pl.ds(rowbase, ROWS_PT), pl.ds(D, DP - D)],
                            outc.at[r, pl.ds(rowbase, ROWS_PT)])
            pltpu.sync_copy(zeros, acc.at[pl.ds(rowbase, ROWS_PT)])
            plsc.subcore_barrier()


@functools.cache
def _make_sc_agg():
    return pl.kernel(
        _sc_agg_body,
        out_type=(jax.ShapeDtypeStruct((R, NPAD, D), jnp.float32),
                  jax.ShapeDtypeStruct((R, NPAD, DP - D), jnp.float32)),
        mesh=plsc.VectorSubcoreMesh(core_axis_name="c", subcore_axis_name="s",
                                    num_cores=NC, num_subcores=NS),
        scratch_types=[
            pltpu.MemorySpace.VMEM_SHARED((NPAD, DP), jnp.float32),
            pltpu.VMEM((MC,), jnp.int32),
            pltpu.VMEM((CAP,), jnp.int32),
            pltpu.VMEM((CAP,), jnp.int32),
            pltpu.VMEM((KF * G, DP), jnp.float32),
            pltpu.VMEM((G, DP), jnp.float32),
            pltpu.SemaphoreType.DMA,
            pltpu.SemaphoreType.DMA,
            pltpu.SemaphoreType.DMA,
            pltpu.SemaphoreType.DMA,
        ],
        compiler_params=pltpu.CompilerParams(needs_layout_passes=False,
                                             use_tc_tiling_on_sc=False),
    )


BN = 2000


def _tc_layer_body(first, aggs_ref, aggc_ref, hin_ref, root_ref, w_ref,
                   b_ref, g_ref, beta_ref, out_ref):
    hin = hin_ref[:, :D]
    acc = jnp.dot(hin, root_ref[...], preferred_element_type=jnp.float32)
    acc = acc + b_ref[...]
    cnt = aggc_ref[:, :, 0:1]
    inv = 1.0 / jnp.maximum(cnt, 1.0)
    for r in range(R):
        mean = aggs_ref[r] * inv[r]
        acc = acc + jnp.dot(mean, w_ref[r], preferred_element_type=jnp.float32)
    if first:
        acc = jnp.where(acc > 0, acc, 0.2 * acc)
    mu = jnp.mean(acc, axis=1, keepdims=True)
    var = jnp.mean((acc - mu) ** 2, axis=1, keepdims=True)
    h = (acc - mu) * lax.rsqrt(var + EPS) * g_ref[...] + beta_ref[...]
    if first:
        hp = jnp.pad(h, ((0, 0), (0, DP - D)))
        lane = lax.broadcasted_iota(jnp.int32, (BN, DP), 1)
        out_ref[...] = hp + jnp.where(lane == D, 1.0, 0.0)
    else:
        out_ref[...] = h


def _tc_layer(first, aggs, aggc, hin, root, w, b, g, beta):
    out_w = DP if first else D
    return pl.pallas_call(
        functools.partial(_tc_layer_body, first),
        grid=(N // BN,),
        in_specs=[
            pl.BlockSpec((R, BN, D), lambda i: (0, i, 0)),
            pl.BlockSpec((R, BN, DP - D), lambda i: (0, i, 0)),
            pl.BlockSpec((BN, DP), lambda i: (i, 0)),
            pl.BlockSpec((D, D), lambda i: (0, 0)),
            pl.BlockSpec((R, D, D), lambda i: (0, 0, 0)),
            pl.BlockSpec((1, D), lambda i: (0, 0)),
            pl.BlockSpec((1, D), lambda i: (0, 0)),
            pl.BlockSpec((1, D), lambda i: (0, 0)),
        ],
        out_specs=pl.BlockSpec((BN, out_w), lambda i: (i, 0)),
        out_shape=jax.ShapeDtypeStruct((N, out_w), jnp.float32),
    )(aggs, aggc, hin, root, w, b, g, beta)


def kernel(x, edge_index, edge_type, W1, root1, b1, g1, beta1,
           W2, root2, b2, g2, beta2):
    src = edge_index[0]
    dst = edge_index[1]
    packed = (src << 17) | (dst << 3) | edge_type
    ones_col = jnp.concatenate(
        [jnp.zeros((N, D), jnp.float32),
         jnp.ones((N, 1), jnp.float32),
         jnp.zeros((N, DP - D - 1), jnp.float32)], axis=1)
    xpad = jnp.pad(x, ((0, 0), (0, DP - D))) + ones_col
    zeros = jnp.zeros((ROWS_PT, DP), jnp.float32)

    _sc_agg = _make_sc_agg()
    aggs1, aggc1 = _sc_agg(xpad, packed, zeros)
    h1pad = _tc_layer(True, aggs1, aggc1, xpad, root1, W1, b1.reshape(1, D),
                      g1.reshape(1, D), beta1.reshape(1, D))
    aggs2, aggc2 = _sc_agg(h1pad, packed, zeros)
    h2 = _tc_layer(False, aggs2, aggc2, h1pad, root2, W2, b2.reshape(1, D),
                   g2.reshape(1, D), beta2.reshape(1, D))
    return h2


# speedup vs baseline: 2.0622x; 1.0663x over previous
import functools

import jax
import jax.numpy as jnp
from jax import lax
from jax.experimental import pallas as pl
from jax.experimental.pallas import tpu as pltpu
from jax.experimental.pallas import tpu_sc as plsc

N = 10000
E = 320000
D = 128
R = 8
EPS = 1e-5

DP = 144
NC = 2
NS = 16
EPT = E // NS
MC = 4000
NMC = EPT // MC
RPC = R // NC
G = 32
ROWS_PT = 641
NPAD = NS * ROWS_PT
TRASH = N
TP = MC + 2 * G
CAP = TP + 16


@functools.cache
def _make_sc_agg(dp, with_counts):
    def body(hpad, packed, zeros, *refs):
        if with_counts:
            outs, outc, acc, metam, gidx, sidx, rows0, gsem, ssem = refs
        else:
            outs, acc, metam, gidx, sidx, rows0, gsem, ssem = refs
        c = lax.axis_index("c")
        s = lax.axis_index("s")

        pltpu.sync_copy(zeros, acc.at[pl.ds(s * ROWS_PT, ROWS_PT)])
        plsc.subcore_barrier()

        @pl.loop(0, RPC)
        def _per_relation(rl):
            r = c * RPC + rl

            def per_chunk(m, _):
                with jax.named_scope("meta_dma"):
                    base = s * EPT + m * MC
                    pltpu.sync_copy(packed.at[pl.ds(base, MC)], metam)

                def scan_group(i, off):
                    p = metam[pl.ds(i * 16, 16)]
                    t = p & 7
                    dv = (p >> 3) & 16383
                    sv = p >> 17
                    d = t ^ r
                    mi = 1 - ((d + 7) >> 3)
                    cum = plsc.cumsum(mi)
                    pos = (off + cum - 1 - TP) * mi + TP
                    plsc.store_scatter(gidx, [pos], sv)
                    plsc.store_scatter(sidx, [pos], dv)
                    return off + cum[15]

                with jax.named_scope("scan"):
                    n_c = lax.fori_loop(0, MC // 16, scan_group, jnp.int32(0))

                pad_end = ((n_c + G - 1) // G) * G

                def pad_group(k, _):
                    j = n_c + k * 16
                    gidx[pl.ds(j, 16)] = jnp.zeros((16,), jnp.int32)
                    sidx[pl.ds(j, 16)] = (TRASH + s * 16 +
                                          lax.iota(jnp.int32, 16))
                    return 0

                lax.fori_loop(0, (pad_end - n_c + 15) // 16, pad_group, 0)

                def move_chunk(q, _):
                    base = q * G
                    pltpu.async_copy(hpad.at[gidx.at[pl.ds(base, G)]],
                                     rows0, gsem).wait()
                    pltpu.sync_copy(rows0,
                                    acc.at[sidx.at[pl.ds(base, G)]],
                                    add=True)
                    return 0

                with jax.named_scope("move"):
                    lax.fori_loop(0, pad_end // G, move_chunk, 0)
                return 0

            lax.fori_loop(0, NMC, per_chunk, 0)
            plsc.subcore_barrier()

            with jax.named_scope("flush"):
                rowbase = s * ROWS_PT
                if with_counts:
                    pltpu.sync_copy(
                        acc.at[pl.ds(rowbase, ROWS_PT), pl.ds(0, D)],
                        outs.at[r, pl.ds(rowbase, ROWS_PT)])
                    pltpu.sync_copy(
                        acc.at[pl.ds(rowbase, ROWS_PT), pl.ds(D, dp - D)],
                        outc.at[r, pl.ds(rowbase, ROWS_PT)])
                else:
                    pltpu.sync_copy(acc.at[pl.ds(rowbase, ROWS_PT)],
                                    outs.at[r, pl.ds(rowbase, ROWS_PT)])
                pltpu.sync_copy(zeros, acc.at[pl.ds(rowbase, ROWS_PT)])
                plsc.subcore_barrier()

    out_type = jax.ShapeDtypeStruct((R, NPAD, D), jnp.float32)
    if with_counts:
        out_type = (out_type,
                    jax.ShapeDtypeStruct((R, NPAD, dp - D), jnp.float32))
    return pl.kernel(
        body,
        out_type=out_type,
        mesh=plsc.VectorSubcoreMesh(core_axis_name="c", subcore_axis_name="s",
                                    num_cores=NC, num_subcores=NS),
        scratch_types=[
            pltpu.MemorySpace.VMEM_SHARED((NPAD, dp), jnp.float32),
            pltpu.VMEM((MC,), jnp.int32),
            pltpu.VMEM((CAP,), jnp.int32),
            pltpu.VMEM((CAP,), jnp.int32),
            pltpu.VMEM((G, dp), jnp.float32),
            pltpu.SemaphoreType.DMA,
            pltpu.SemaphoreType.DMA,
        ],
        compiler_params=pltpu.CompilerParams(needs_layout_passes=False,
                                             use_tc_tiling_on_sc=False),
    )


BN = 2000


def _tc_layer_body(first, aggs_ref, aggc_ref, hin_ref, root_ref, w_ref,
                   b_ref, g_ref, beta_ref, out_ref):
    acc = jnp.dot(hin_ref[...], root_ref[...],
                  preferred_element_type=jnp.float32)
    acc = acc + b_ref[...]
    cnt = aggc_ref[:, :, 0:1]
    inv = 1.0 / jnp.maximum(cnt, 1.0)
    for r in range(R):
        mean = aggs_ref[r] * inv[r]
        acc = acc + jnp.dot(mean, w_ref[r], preferred_element_type=jnp.float32)
    if first:
        acc = jnp.where(acc > 0, acc, 0.2 * acc)
    mu = jnp.mean(acc, axis=1, keepdims=True)
    var = jnp.mean((acc - mu) ** 2, axis=1, keepdims=True)
    out_ref[...] = ((acc - mu) * lax.rsqrt(var + EPS) * g_ref[...]
                    + beta_ref[...])


def _tc_layer(first, aggs, aggc, hin, root, w, b, g, beta):
    return pl.pallas_call(
        functools.partial(_tc_layer_body, first),
        grid=(N // BN,),
        in_specs=[
            pl.BlockSpec((R, BN, D), lambda i: (0, i, 0)),
            pl.BlockSpec((R, BN, DP - D), lambda i: (0, i, 0)),
            pl.BlockSpec((BN, D), lambda i: (i, 0)),
            pl.BlockSpec((D, D), lambda i: (0, 0)),
            pl.BlockSpec((R, D, D), lambda i: (0, 0, 0)),
            pl.BlockSpec((1, D), lambda i: (0, 0)),
            pl.BlockSpec((1, D), lambda i: (0, 0)),
            pl.BlockSpec((1, D), lambda i: (0, 0)),
        ],
        out_specs=pl.BlockSpec((BN, D), lambda i: (i, 0)),
        out_shape=jax.ShapeDtypeStruct((N, D), jnp.float32),
    )(aggs, aggc, hin, root, w, b, g, beta)


def kernel(x, edge_index, edge_type, W1, root1, b1, g1, beta1,
           W2, root2, b2, g2, beta2):
    src = edge_index[0]
    dst = edge_index[1]
    packed = (src << 17) | (dst << 3) | edge_type
    ones_col = jnp.concatenate(
        [jnp.zeros((N, D), jnp.float32),
         jnp.ones((N, 1), jnp.float32),
         jnp.zeros((N, DP - D - 1), jnp.float32)], axis=1)
    xpad = jnp.pad(x, ((0, 0), (0, DP - D))) + ones_col
    zeros1 = jnp.zeros((ROWS_PT, DP), jnp.float32)
    zeros2 = jnp.zeros((ROWS_PT, D), jnp.float32)

    aggs1, aggc1 = _make_sc_agg(DP, True)(xpad, packed, zeros1)
    h1 = _tc_layer(True, aggs1, aggc1, x, root1, W1, b1.reshape(1, D),
                   g1.reshape(1, D), beta1.reshape(1, D))
    aggs2 = _make_sc_agg(D, False)(h1, packed, zeros2)
    h2 = _tc_layer(False, aggs2, aggc1, h1, root2, W2, b2.reshape(1, D),
                   g2.reshape(1, D), beta2.reshape(1, D))
    return h2
